# Initial kernel scaffold; baseline (speedup 1.0000x reference)
import jax
import jax.numpy as jnp
from jax.experimental import pallas as pl


def kernel(x_sequence, edge_index, sensor_idx, W1, b1, W2, b2, Wih0, Whh0, bih0, bhh0, Wih1, Whh1, bih1, bhh1, fcW, fcb):
    raise NotImplementedError("write your pallas kernel here")


# speedup vs baseline: 7.6740x; 7.6740x over previous
import functools

import jax
import jax.numpy as jnp
from jax.experimental import pallas as pl
from jax.experimental.pallas import tpu as pltpu


def _h2_of(a, w1_ref, b1_ref, w2_ref, H):
    acc = jnp.zeros_like(a)
    for k in range(H):
        w1k = w1_ref[0:1, k:k + 1]
        b1k = b1_ref[0:1, k:k + 1]
        w2k = w2_ref[0:1, k:k + 1]
        acc = acc + w2k * jnp.maximum(a * w1k + b1k, 0.0)
    return acc


def _tc1_body(x0_ref, a_ref, xall_ref, w1_ref, b1_ref, w2_ref,
              h20_ref, h2x_ref, *, H):
    a0 = jax.lax.dot_general(
        x0_ref[...], a_ref[...], (((1,), (1,)), ((), ())),
        preferred_element_type=jnp.float32)
    h20_ref[...] = _h2_of(a0, w1_ref, b1_ref, w2_ref, H)
    h2x_ref[...] = _h2_of(xall_ref[...], w1_ref, b1_ref, w2_ref, H)


def _tc1(x0, A, xall, w1, b1, w2):
    T, NN = x0.shape[0], A.shape[0]
    H = w1.shape[1]
    return pl.pallas_call(
        functools.partial(_tc1_body, H=H),
        out_shape=(jax.ShapeDtypeStruct((T, NN), jnp.float32),
                   jax.ShapeDtypeStruct(xall.shape, jnp.float32)),
    )(x0, A, xall, w1, b1, w2)


def _gru_unrolled(gi, Whh_t, bhh, B, T, HG):
    h = jnp.zeros((B, HG), jnp.float32)
    outs = []
    for t in range(T):
        gh = jnp.dot(h, Whh_t, preferred_element_type=jnp.float32) + bhh
        g = gi[:, t, :]
        r = jax.nn.sigmoid(g[:, :HG] + gh[:, :HG])
        z = jax.nn.sigmoid(g[:, HG:2 * HG] + gh[:, HG:2 * HG])
        n = jnp.tanh(g[:, 2 * HG:] + r * gh[:, 2 * HG:])
        h = (1.0 - z) * n + z * h
        outs.append(h)
    return outs


def _tc2_body(agg_ref, h20_ref, dinv_ref, h2x_ref, wih0_ref, wsens_ref,
              b1_ref, w2_ref, b2_ref, bih0_ref, whh0_ref, bhh0_ref,
              wih1_ref, whh1_ref, bih1_ref, bhh1_ref, fcw_ref, fcb_ref,
              out_ref, *, B, T, NN, S, HG, H):
    b2 = b2_ref[0:1, 0:1]
    dinv = dinv_ref[...]
    out2_0 = agg_ref[...] + dinv * dinv * h20_ref[...] + b2

    wih0 = wih0_ref[...]
    gi_b0 = jax.lax.dot_general(
        out2_0, wih0, (((1,), (1,)), ((), ())),
        preferred_element_type=jnp.float32)

    c0 = jnp.zeros((1, 1), jnp.float32)
    for k in range(H):
        c0 = c0 + w2_ref[0:1, k:k + 1] * jnp.maximum(b1_ref[0:1, k:k + 1], 0.0)
    cc = c0 + b2

    rs = jnp.sum(wih0, axis=1)[None, :]
    h2b = h2x_ref[...][T:, :]
    gi_rest = jax.lax.dot_general(
        h2b + (b2 - cc), wsens_ref[...], (((1,), (1,)), ((), ())),
        preferred_element_type=jnp.float32) + cc * rs

    bih0 = bih0_ref[...]
    gi0 = jnp.concatenate([gi_b0, gi_rest], axis=0) + bih0
    gi0 = gi0.reshape(B, T, 3 * HG)

    o0 = _gru_unrolled(gi0, whh0_ref[...].T, bhh0_ref[...], B, T, HG)
    gi1 = jnp.concatenate([o[:, None, :] for o in o0], axis=1)
    gi1 = jax.lax.dot_general(
        gi1.reshape(B * T, HG), wih1_ref[...], (((1,), (1,)), ((), ())),
        preferred_element_type=jnp.float32).reshape(B, T, 3 * HG) + bih1_ref[...]
    o1 = _gru_unrolled(gi1, whh1_ref[...].T, bhh1_ref[...], B, T, HG)
    last = o1[-1]

    out_ref[...] = jax.lax.dot_general(
        last, fcw_ref[...], (((1,), (1,)), ((), ())),
        preferred_element_type=jnp.float32) + fcb_ref[...]


def _tc2(agg, h2_0, dinv, h2x, Wih0, Wsens, b1, w2, b2, bih0, Whh0, bhh0,
         Wih1, Whh1, bih1, bhh1, fcW, fcb):
    T, NN = agg.shape
    B = h2x.shape[0] // T
    S = h2x.shape[1]
    HG = Whh0.shape[1]
    H = w2.shape[1]
    return pl.pallas_call(
        functools.partial(_tc2_body, B=B, T=T, NN=NN, S=S, HG=HG, H=H),
        out_shape=jax.ShapeDtypeStruct((B, NN), jnp.float32),
    )(agg, h2_0, dinv, h2x, Wih0, Wsens, b1, w2, b2, bih0, Whh0, bhh0,
      Wih1, Whh1, bih1, bhh1, fcW, fcb)


def kernel(x_sequence, edge_index, sensor_idx, W1, b1, W2, b2,
           Wih0, Whh0, bih0, bhh0, Wih1, Whh1, bih1, bhh1, fcW, fcb):
    B, T, S = x_sequence.shape
    NN = fcb.shape[0]
    src = edge_index[0]
    dst = edge_index[1]

    deg = jnp.zeros((NN,), jnp.float32).at[dst].add(1.0) + 1.0
    dinv = jax.lax.rsqrt(deg)
    norm = dinv[src] * dinv[dst]

    inv = jnp.full((NN,), -1, jnp.int32).at[sensor_idx].set(
        jnp.arange(S, dtype=jnp.int32))
    k_of_src = inv[src]
    smask = k_of_src >= 0
    A = jnp.zeros((NN, S), jnp.float32).at[
        dst, jnp.where(smask, k_of_src, 0)].add(jnp.where(smask, norm, 0.0))
    A = A.at[sensor_idx, jnp.arange(S)].add(dinv[sensor_idx] ** 2)

    x0 = x_sequence[0]
    xall = x_sequence.reshape(B * T, S)
    h2_0, h2x = _tc1(x0, A, xall, W1, b1.reshape(1, -1), W2.reshape(1, -1))

    agg = jnp.zeros((T, NN), jnp.float32).at[:, dst].add(
        norm[None, :] * h2_0[:, src])

    Wsens = jnp.take(Wih0, sensor_idx, axis=1)
    return _tc2(agg, h2_0, dinv.reshape(1, -1), h2x, Wih0, Wsens,
                b1.reshape(1, -1), W2.reshape(1, -1), b2.reshape(1, 1),
                bih0.reshape(1, -1), Whh0, bhh0.reshape(1, -1),
                Wih1, Whh1, bih1.reshape(1, -1), bhh1.reshape(1, -1),
                fcW, fcb.reshape(1, -1))


# speedup vs baseline: 177.0074x; 23.0660x over previous
import functools

import jax
import jax.numpy as jnp
from jax import lax
from jax.experimental import pallas as pl
from jax.experimental.pallas import tpu as pltpu
from jax.experimental.pallas import tpu_sc as plsc

NC = 2
NS = 16
LANES = 16

_HIGH = jax.lax.Precision.HIGHEST


def _mesh():
    return plsc.VectorSubcoreMesh(
        core_axis_name="c", subcore_axis_name="s", num_cores=NC,
        num_subcores=NS)


def _sc1_body(src_hbm, dst_hbm, sens_hbm, degp_hbm, cpart_hbm,
              srcb, dstb, invb, degb, flat1, flat2, ones2, zb, sensb, c_sh,
              *, NN, S, EPT, CH, NWIN, WIN, DEGN):
    c = lax.axis_index("c")
    s = lax.axis_index("s")
    base = s * EPT
    SH = S // 2
    CSZ = NN * SH
    CT = CSZ // NS

    neg1 = jnp.full((LANES,), -1, jnp.int32)
    zi = jnp.zeros((LANES,), jnp.int32)
    zf = jnp.zeros((LANES,), jnp.float32)
    onef = jnp.ones((LANES,), jnp.float32)

    def ini(i, _):
        invb[pl.ds(i * LANES, LANES)] = neg1
        return 0
    lax.fori_loop(0, (NN + LANES) // LANES, ini, 0)

    def ini2(i, _):
        flat1[pl.ds(i * LANES, LANES)] = zi
        return 0
    lax.fori_loop(0, (NWIN * WIN) // LANES, ini2, 0)

    def ini3(i, _):
        degb[pl.ds(i * LANES, LANES)] = zf
        return 0
    lax.fori_loop(0, DEGN // LANES, ini3, 0)

    def ini4(i, _):
        zb[pl.ds(i * LANES, LANES)] = zf
        return 0
    lax.fori_loop(0, 4000 // LANES, ini4, 0)

    for g in range(WIN // LANES):
        ones2[0, pl.ds(g * LANES, LANES)] = onef

    for j in range(CT // 4000):
        pltpu.sync_copy(zb, c_sh.at[pl.ds(s * CT + j * 4000, 4000)])

    pltpu.sync_copy(sens_hbm, sensb)
    for g in range(S // LANES):
        sidx = sensb[pl.ds(g * LANES, LANES)]
        plsc.store_scatter(invb, [sidx],
                           lax.iota(jnp.int32, LANES) + g * LANES)

    plsc.subcore_barrier()

    koff = c * SH

    def chunk(i, off):
        pltpu.sync_copy(src_hbm.at[pl.ds(base + i * CH, CH)], srcb)
        pltpu.sync_copy(dst_hbm.at[pl.ds(base + i * CH, CH)], dstb)

        def grp(g, off):
            s16 = srcb[pl.ds(g * LANES, LANES)]
            d16 = dstb[pl.ds(g * LANES, LANES)]
            plsc.addupdate_scatter(degb, [d16], onef)
            k16 = plsc.load_gather(invb, [s16]) - koff
            m = (k16 >= 0) & (k16 < SH)
            flat = d16 * SH + k16
            plsc.store_compressed(flat1.at[pl.ds(off, LANES)], flat, mask=m)
            return off + jnp.sum(jnp.where(m, 1, 0))
        return lax.fori_loop(0, CH // LANES, grp, off)
    cnt = lax.fori_loop(0, EPT // CH, chunk, jnp.int32(0))

    nwin = (cnt + (WIN - 1)) // WIN

    def towin(w, _):
        for g in range(WIN // LANES):
            flat2[w, pl.ds(g * LANES, LANES)] = \
                flat1[pl.ds(w * WIN + g * LANES, LANES)]
        return 0
    lax.fori_loop(0, nwin, towin, 0)

    tail = cnt - (nwin - 1) * WIN
    for g in range(WIN // LANES):
        ones2[1, pl.ds(g * LANES, LANES)] = jnp.where(
            lax.iota(jnp.int32, LANES) + g * LANES < tail, 1.0, 0.0)

    def scat(w, _):
        row = jnp.where(w < nwin - 1, 0, 1)
        pltpu.sync_copy(ones2.at[row], c_sh.at[flat2.at[w]], add=True)
        return 0
    lax.fori_loop(0, nwin, scat, 0)

    @pl.when(s == 0)
    def _():
        for g in range(WIN // LANES):
            if g < SH // LANES:
                sidx = sensb[pl.ds(koff + g * LANES, LANES)]
                flat2[0, pl.ds(g * LANES, LANES)] = \
                    sidx * SH + lax.iota(jnp.int32, LANES) + g * LANES
                ones2[1, pl.ds(g * LANES, LANES)] = onef
            else:
                flat2[0, pl.ds(g * LANES, LANES)] = zi
                ones2[1, pl.ds(g * LANES, LANES)] = zf
        pltpu.sync_copy(ones2.at[1], c_sh.at[flat2.at[0]], add=True)

    plsc.subcore_barrier()

    @pl.when(c == 0)
    def _():
        pltpu.sync_copy(degb, degp_hbm.at[pl.ds(s * DEGN, DEGN)])
    for j in range(CT // 4000):
        pltpu.sync_copy(c_sh.at[pl.ds(s * CT + j * 4000, 4000)], zb)
        pltpu.sync_copy(zb, cpart_hbm.at[pl.ds(c * CSZ + s * CT + j * 4000,
                                               4000)])


def _sc1(srcp, dstp, sensor_idx, NN, S):
    E = srcp.shape[0]
    EPT = E // NS
    CH = 1024
    WIN = 128
    NWIN = EPT // WIN
    DEGN = NN + LANES
    body = functools.partial(_sc1_body, NN=NN, S=S, EPT=EPT, CH=CH,
                             NWIN=NWIN, WIN=WIN, DEGN=DEGN)
    f = pl.kernel(
        body,
        out_type=(jax.ShapeDtypeStruct((NS * DEGN,), jnp.float32),
                  jax.ShapeDtypeStruct((NN * S,), jnp.float32)),
        mesh=_mesh(),
        compiler_params=pltpu.CompilerParams(needs_layout_passes=False),
        scratch_types=[
            pltpu.VMEM((CH,), jnp.int32),
            pltpu.VMEM((CH,), jnp.int32),
            pltpu.VMEM((NN + LANES,), jnp.int32),
            pltpu.VMEM((DEGN,), jnp.float32),
            pltpu.VMEM((NWIN * WIN,), jnp.int32),
            pltpu.VMEM((NWIN, WIN), jnp.int32),
            pltpu.VMEM((2, WIN), jnp.float32),
            pltpu.VMEM((4000,), jnp.float32),
            pltpu.VMEM((S,), jnp.int32),
            pltpu.VMEM_SHARED((NN * S // 2,), jnp.float32),
        ],
    )
    return f(srcp, dstp, sensor_idx)


def _sc2_body(src_hbm, dst_hbm, utab_hbm, aggp_hbm,
              srcb, dwins, rows, zrows, agg_sh, sem, *, NN, NNP, EPT, CH, WIN):
    c = lax.axis_index("c")
    s = lax.axis_index("s")
    wid = c * NS + s
    nwc = CH // WIN
    RPT = NNP // NS

    zf = jnp.zeros((LANES,), jnp.float32)

    def ini(i, _):
        zrows[i] = zf
        return 0
    lax.fori_loop(0, RPT, ini, 0)
    pltpu.sync_copy(zrows, agg_sh.at[pl.ds(s * RPT, RPT)])
    plsc.subcore_barrier()

    def chunk(i, _):
        base = wid * EPT + i * CH
        pltpu.sync_copy(src_hbm.at[pl.ds(base, CH)], srcb)
        for w in range(nwc):
            pltpu.sync_copy(dst_hbm.at[pl.ds(base + w * WIN, WIN)], dwins[w])
        pltpu.async_copy(utab_hbm.at[srcb], rows, sem).wait()
        for w in range(nwc):
            pltpu.sync_copy(rows.at[pl.ds(w * WIN, WIN)],
                            agg_sh.at[dwins[w]], add=True)
        return 0
    lax.fori_loop(0, EPT // CH, chunk, 0)

    plsc.subcore_barrier()
    pltpu.sync_copy(agg_sh.at[pl.ds(s * RPT, RPT)], zrows)
    pltpu.sync_copy(zrows, aggp_hbm.at[c, s])


def _sc2(srcp, dst2, utab, NN):
    E = srcp.shape[0]
    EPT = E // (NC * NS)
    CH = 1024
    WIN = 128
    NNP = 10240
    body = functools.partial(_sc2_body, NN=NN, NNP=NNP, EPT=EPT, CH=CH,
                             WIN=WIN)
    f = pl.kernel(
        body,
        out_type=jax.ShapeDtypeStruct((NC, NS, NNP // NS, LANES),
                                      jnp.float32),
        mesh=_mesh(),
        compiler_params=pltpu.CompilerParams(needs_layout_passes=False,
                                             use_tc_tiling_on_sc=False),
        scratch_types=[
            pltpu.VMEM((CH,), jnp.int32),
            [pltpu.VMEM((WIN,), jnp.int32)
             for _ in range(CH // WIN)],
            pltpu.VMEM((CH, LANES), jnp.float32),
            pltpu.VMEM((NNP // NS, LANES), jnp.float32),
            pltpu.VMEM_SHARED((NNP, LANES), jnp.float32),
            pltpu.SemaphoreType.DMA,
        ],
    )
    return f(srcp, dst2, utab)


def _h2_of(a, w1_ref, b1_ref, w2_ref, H):
    acc = jnp.zeros_like(a)
    for k in range(H):
        w1k = w1_ref[0:1, k:k + 1]
        b1k = b1_ref[0:1, k:k + 1]
        w2k = w2_ref[0:1, k:k + 1]
        acc = acc + w2k * jnp.maximum(a * w1k + b1k, 0.0)
    return acc


def _tc1_body(degp_ref, c0_ref, c1_ref, xs0_ref, xs1_ref, xall_ref,
              w1_ref, b1_ref, w2_ref,
              utab_ref, h2x_ref, dinv_ref, *, NN, H):
    deg = jnp.sum(degp_ref[...], axis=0, keepdims=True) + 1.0
    dinv = lax.rsqrt(deg)[:, :NN]
    dinv_ref[...] = dinv
    dcol = dinv.reshape(NN, 1)
    a0t = dcol * (jax.lax.dot_general(
        c0_ref[...], xs0_ref[...], (((1,), (0,)), ((), ())),
        precision=_HIGH, preferred_element_type=jnp.float32) +
        jax.lax.dot_general(
        c1_ref[...], xs1_ref[...], (((1,), (0,)), ((), ())),
        precision=_HIGH, preferred_element_type=jnp.float32))
    utab = dcol * _h2_of(a0t, w1_ref, b1_ref, w2_ref, H)
    utab_ref[...] = jnp.concatenate(
        [utab, jnp.zeros((8, LANES), jnp.float32)], axis=0)
    h2x_ref[...] = _h2_of(xall_ref[...], w1_ref, b1_ref, w2_ref, H)


def _tc1(degp, c0, c1, xs0, xs1, xall, w1, b1, w2):
    NN = c0.shape[0]
    H = w1.shape[1]
    return pl.pallas_call(
        functools.partial(_tc1_body, NN=NN, H=H),
        out_shape=(jax.ShapeDtypeStruct((NN + 8, LANES), jnp.float32),
                   jax.ShapeDtypeStruct(xall.shape, jnp.float32),
                   jax.ShapeDtypeStruct((1, NN), jnp.float32)),
    )(degp, c0, c1, xs0, xs1, xall, w1, b1, w2)


def _gru_unrolled(gi, Whh_t, bhh, B, T, HG):
    h = jnp.zeros((B, HG), jnp.float32)
    outs = []
    for t in range(T):
        gh = jnp.dot(h, Whh_t, preferred_element_type=jnp.float32) + bhh
        g = gi[:, t, :]
        r = jax.nn.sigmoid(g[:, :HG] + gh[:, :HG])
        z = jax.nn.sigmoid(g[:, HG:2 * HG] + gh[:, HG:2 * HG])
        n = jnp.tanh(g[:, 2 * HG:] + r * gh[:, 2 * HG:])
        h = (1.0 - z) * n + z * h
        outs.append(h)
    return outs


def _tc2_body(agg0_ref, agg1_ref, utab_ref, dinv_ref, h2x_ref, wih0_ref, wsens_ref,
              b1_ref, w2_ref, b2_ref, bih0_ref, whh0_ref, bhh0_ref,
              wih1_ref, whh1_ref, bih1_ref, bhh1_ref, fcw_ref, fcb_ref,
              out_ref, *, B, T, NN, S, HG, H):
    b2 = b2_ref[0:1, 0:1]
    utab = utab_ref[...][:NN, :]
    aggu = agg0_ref[...] + agg1_ref[...]
    dcol = dinv_ref[...].reshape(NN, 1)
    out2t = dcol * (aggu + utab) + b2

    wih0 = wih0_ref[...]
    git = jax.lax.dot_general(
        wih0, out2t, (((1,), (0,)), ((), ())),
        precision=_HIGH, preferred_element_type=jnp.float32)
    gi_b0 = git.T[:T, :]

    c0 = jnp.zeros((1, 1), jnp.float32)
    for k in range(H):
        c0 = c0 + w2_ref[0:1, k:k + 1] * jnp.maximum(b1_ref[0:1, k:k + 1], 0.0)
    cc = c0 + b2

    rs = jnp.sum(wih0, axis=1)[None, :]
    h2b = h2x_ref[...][T:, :]
    gi_rest = jax.lax.dot_general(
        h2b + (b2 - cc), wsens_ref[...], (((1,), (1,)), ((), ())),
        precision=_HIGH, preferred_element_type=jnp.float32) + cc * rs

    bih0 = bih0_ref[...]
    gi0 = jnp.concatenate([gi_b0, gi_rest], axis=0) + bih0
    gi0 = gi0.reshape(B, T, 3 * HG)

    o0 = _gru_unrolled(gi0, whh0_ref[...].T, bhh0_ref[...], B, T, HG)
    gi1 = jnp.concatenate([o[:, None, :] for o in o0], axis=1)
    gi1 = jax.lax.dot_general(
        gi1.reshape(B * T, HG), wih1_ref[...], (((1,), (1,)), ((), ())),
        precision=_HIGH,
        preferred_element_type=jnp.float32).reshape(B, T, 3 * HG) + bih1_ref[...]
    o1 = _gru_unrolled(gi1, whh1_ref[...].T, bhh1_ref[...], B, T, HG)
    last = o1[-1]

    out_ref[...] = jax.lax.dot_general(
        last, fcw_ref[...], (((1,), (1,)), ((), ())),
        precision=_HIGH, preferred_element_type=jnp.float32) + fcb_ref[...]


def _tc2(agg0, agg1, utab, dinv, h2x, Wih0, Wsens, b1, w2, b2, bih0, Whh0, bhh0,
         Wih1, Whh1, bih1, bhh1, fcW, fcb):
    NN = Wih0.shape[1]
    T = 12
    B = h2x.shape[0] // T
    S = h2x.shape[1]
    HG = Whh0.shape[1]
    H = w2.shape[1]
    return pl.pallas_call(
        functools.partial(_tc2_body, B=B, T=T, NN=NN, S=S, HG=HG, H=H),
        out_shape=jax.ShapeDtypeStruct((B, NN), jnp.float32),
    )(agg0, agg1, utab, dinv, h2x, Wih0, Wsens, b1, w2, b2, bih0, Whh0, bhh0,
      Wih1, Whh1, bih1, bhh1, fcW, fcb)


def kernel(x_sequence, edge_index, sensor_idx, W1, b1, W2, b2,
           Wih0, Whh0, bih0, bhh0, Wih1, Whh1, bih1, bhh1, fcW, fcb):
    B, T, S = x_sequence.shape
    NN = fcb.shape[0]
    E = edge_index.shape[1]

    EPAD = ((E + NC * NS * 2048 - 1) // (NC * NS * 2048)) * (NC * NS * 2048)
    srcp = jnp.concatenate(
        [edge_index[0], jnp.full((EPAD - E,), NN, jnp.int32)])
    dstp = jnp.concatenate(
        [edge_index[1], jnp.full((EPAD - E,), NN, jnp.int32)])
    degp, cpart = _sc1(srcp, dstp, sensor_idx, NN, S)
    degp = degp.reshape(NS, NN + LANES)

    degS = jnp.sum(degp[:, sensor_idx], axis=0) + 1.0
    dinvS = lax.rsqrt(degS)
    x0p = jnp.pad(x_sequence[0], ((0, LANES - T), (0, 0)))
    xs = x0p.T * dinvS[:, None]
    xall = x_sequence.reshape(B * T, S)

    cpart = cpart.reshape(NC, NN, S // 2)
    utab, h2x, dinv = _tc1(degp, cpart[0], cpart[1],
                           xs[:S // 2], xs[S // 2:], xall,
                           W1, b1.reshape(1, -1), W2.reshape(1, -1))

    aggp = _sc2(srcp, dstp, utab, NN)
    aggp = aggp.reshape(NC, 10240, LANES)[:, :NN, :]

    Wsens = jnp.take(Wih0, sensor_idx, axis=1)
    return _tc2(aggp[0], aggp[1], utab, dinv.reshape(NN, 1), h2x, Wih0, Wsens,
                b1.reshape(1, -1), W2.reshape(1, -1), b2.reshape(1, 1),
                bih0.reshape(1, -1), Whh0, bhh0.reshape(1, -1),
                Wih1, Whh1, bih1.reshape(1, -1), bhh1.reshape(1, -1),
                fcW, fcb.reshape(1, -1))


# speedup vs baseline: 198.1942x; 1.1197x over previous
import functools

import jax
import jax.numpy as jnp
from jax import lax
from jax.experimental import pallas as pl
from jax.experimental.pallas import tpu as pltpu
from jax.experimental.pallas import tpu_sc as plsc

NC = 2
NS = 16
LANES = 16

_HIGH = jax.lax.Precision.HIGHEST


def _mesh():
    return plsc.VectorSubcoreMesh(
        core_axis_name="c", subcore_axis_name="s", num_cores=NC,
        num_subcores=NS)


def _sc1_body(src_hbm, dst_hbm, sens_hbm, degp_hbm, cpart_hbm,
              srcb, dstb, invb, degb, flat1, flat2, ones2, zb, sensb, c_sh,
              *, NN, S, EPT, CH, NWIN, WIN, DEGN):
    c = lax.axis_index("c")
    s = lax.axis_index("s")
    base = s * EPT
    SH = S // 2
    CSZ = NN * SH
    CT = CSZ // NS

    neg1 = jnp.full((LANES,), -1, jnp.int32)
    zi = jnp.zeros((LANES,), jnp.int32)
    zf = jnp.zeros((LANES,), jnp.float32)
    onef = jnp.ones((LANES,), jnp.float32)

    def ini(i, _):
        invb[pl.ds(i * LANES, LANES)] = neg1
        return 0
    lax.fori_loop(0, (NN + LANES) // LANES, ini, 0)

    def ini2(i, _):
        flat1[pl.ds(i * LANES, LANES)] = zi
        return 0
    lax.fori_loop(0, (NWIN * WIN) // LANES, ini2, 0)

    def ini3(i, _):
        degb[pl.ds(i * LANES, LANES)] = zf
        return 0
    lax.fori_loop(0, DEGN // LANES, ini3, 0)

    def ini4(i, _):
        zb[pl.ds(i * LANES, LANES)] = zf
        return 0
    lax.fori_loop(0, 4000 // LANES, ini4, 0)

    for g in range(WIN // LANES):
        ones2[0, pl.ds(g * LANES, LANES)] = onef

    for j in range(CT // 4000):
        pltpu.sync_copy(zb, c_sh.at[pl.ds(s * CT + j * 4000, 4000)])

    pltpu.sync_copy(sens_hbm, sensb)
    for g in range(S // LANES):
        sidx = sensb[pl.ds(g * LANES, LANES)]
        plsc.store_scatter(invb, [sidx],
                           lax.iota(jnp.int32, LANES) + g * LANES)

    plsc.subcore_barrier()

    koff = c * SH

    def chunk(i, off):
        pltpu.sync_copy(src_hbm.at[pl.ds(base + i * CH, CH)], srcb)
        pltpu.sync_copy(dst_hbm.at[pl.ds(base + i * CH, CH)], dstb)

        def grp(g, off):
            s16 = srcb[pl.ds(g * LANES, LANES)]
            d16 = dstb[pl.ds(g * LANES, LANES)]
            plsc.addupdate_scatter(degb, [d16], onef)
            k16 = plsc.load_gather(invb, [s16]) - koff
            m = (k16 >= 0) & (k16 < SH)
            flat = d16 * SH + k16
            plsc.store_compressed(flat1.at[pl.ds(off, LANES)], flat, mask=m)
            return off + jnp.sum(jnp.where(m, 1, 0))
        return lax.fori_loop(0, CH // LANES, grp, off)
    cnt = lax.fori_loop(0, EPT // CH, chunk, jnp.int32(0))

    nwin = (cnt + (WIN - 1)) // WIN

    def towin(w, _):
        for g in range(WIN // LANES):
            flat2[w, pl.ds(g * LANES, LANES)] = \
                flat1[pl.ds(w * WIN + g * LANES, LANES)]
        return 0
    lax.fori_loop(0, nwin, towin, 0)

    tail = cnt - (nwin - 1) * WIN
    for g in range(WIN // LANES):
        ones2[1, pl.ds(g * LANES, LANES)] = jnp.where(
            lax.iota(jnp.int32, LANES) + g * LANES < tail, 1.0, 0.0)

    def scat(w, _):
        row = jnp.where(w < nwin - 1, 0, 1)
        pltpu.sync_copy(ones2.at[row], c_sh.at[flat2.at[w]], add=True)
        return 0
    lax.fori_loop(0, nwin, scat, 0)

    @pl.when(s == 0)
    def _():
        for g in range(WIN // LANES):
            if g < SH // LANES:
                sidx = sensb[pl.ds(koff + g * LANES, LANES)]
                flat2[0, pl.ds(g * LANES, LANES)] = \
                    sidx * SH + lax.iota(jnp.int32, LANES) + g * LANES
                ones2[1, pl.ds(g * LANES, LANES)] = onef
            else:
                flat2[0, pl.ds(g * LANES, LANES)] = zi
                ones2[1, pl.ds(g * LANES, LANES)] = zf
        pltpu.sync_copy(ones2.at[1], c_sh.at[flat2.at[0]], add=True)

    plsc.subcore_barrier()

    @pl.when(c == 0)
    def _():
        pltpu.sync_copy(degb, degp_hbm.at[pl.ds(s * DEGN, DEGN)])
    for j in range(CT // 4000):
        pltpu.sync_copy(c_sh.at[pl.ds(s * CT + j * 4000, 4000)], zb)
        pltpu.sync_copy(zb, cpart_hbm.at[pl.ds(c * CSZ + s * CT + j * 4000,
                                               4000)])


def _sc1(srcp, dstp, sensor_idx, NN, S):
    E = srcp.shape[0]
    EPT = E // NS
    CH = 4096
    WIN = 128
    NWIN = EPT // WIN
    DEGN = NN + LANES
    body = functools.partial(_sc1_body, NN=NN, S=S, EPT=EPT, CH=CH,
                             NWIN=NWIN, WIN=WIN, DEGN=DEGN)
    f = pl.kernel(
        body,
        out_type=(jax.ShapeDtypeStruct((NS * DEGN,), jnp.float32),
                  jax.ShapeDtypeStruct((NN * S,), jnp.float32)),
        mesh=_mesh(),
        compiler_params=pltpu.CompilerParams(needs_layout_passes=False),
        scratch_types=[
            pltpu.VMEM((CH,), jnp.int32),
            pltpu.VMEM((CH,), jnp.int32),
            pltpu.VMEM((NN + LANES,), jnp.int32),
            pltpu.VMEM((DEGN,), jnp.float32),
            pltpu.VMEM((NWIN * WIN,), jnp.int32),
            pltpu.VMEM((NWIN, WIN), jnp.int32),
            pltpu.VMEM((2, WIN), jnp.float32),
            pltpu.VMEM((4000,), jnp.float32),
            pltpu.VMEM((S,), jnp.int32),
            pltpu.VMEM_SHARED((NN * S // 2,), jnp.float32),
        ],
    )
    return f(srcp, dstp, sensor_idx)


def _sc2_body(src_hbm, dst_hbm, utab_hbm, aggp_hbm,
              srcb, dstb, rows, zrows, agg_sh, sem, *, NN, NNP, EPT, CH, WIN):
    c = lax.axis_index("c")
    s = lax.axis_index("s")
    wid = c * NS + s
    RPT = NNP // NS

    zf = jnp.zeros((LANES,), jnp.float32)

    def ini(i, _):
        zrows[i] = zf
        return 0
    lax.fori_loop(0, RPT, ini, 0)
    pltpu.sync_copy(zrows, agg_sh.at[pl.ds(s * RPT, RPT)])
    plsc.subcore_barrier()

    def chunk(i, _):
        base = wid * EPT + i * CH
        pltpu.sync_copy(src_hbm.at[pl.ds(base, CH)], srcb)
        pltpu.sync_copy(dst_hbm.at[pl.ds(base, CH)], dstb)
        pltpu.async_copy(utab_hbm.at[srcb], rows, sem).wait()
        pltpu.sync_copy(rows, agg_sh.at[dstb], add=True)
        return 0
    lax.fori_loop(0, EPT // CH, chunk, 0)

    plsc.subcore_barrier()
    pltpu.sync_copy(agg_sh.at[pl.ds(s * RPT, RPT)], zrows)
    pltpu.sync_copy(zrows, aggp_hbm.at[c, s])


def _sc2(srcp, dst2, utab, NN):
    E = srcp.shape[0]
    EPT = E // (NC * NS)
    CH = 2048
    WIN = 128
    NNP = 10240
    body = functools.partial(_sc2_body, NN=NN, NNP=NNP, EPT=EPT, CH=CH,
                             WIN=WIN)
    f = pl.kernel(
        body,
        out_type=jax.ShapeDtypeStruct((NC, NS, NNP // NS, LANES),
                                      jnp.float32),
        mesh=_mesh(),
        compiler_params=pltpu.CompilerParams(needs_layout_passes=False,
                                             use_tc_tiling_on_sc=False),
        scratch_types=[
            pltpu.VMEM((CH,), jnp.int32),
            pltpu.VMEM((CH,), jnp.int32),
            pltpu.VMEM((CH, LANES), jnp.float32),
            pltpu.VMEM((NNP // NS, LANES), jnp.float32),
            pltpu.VMEM_SHARED((NNP, LANES), jnp.float32),
            pltpu.SemaphoreType.DMA,
        ],
    )
    return f(srcp, dst2, utab)


def _pad_body(ei_ref, out_ref, *, E, EPAD, NN):
    out_ref[...] = jnp.concatenate(
        [ei_ref[...], jnp.full((2, EPAD - E), NN, jnp.int32)], axis=1)


def _pad_edges(edge_index, EPAD, NN):
    E = edge_index.shape[1]
    return pl.pallas_call(
        functools.partial(_pad_body, E=E, EPAD=EPAD, NN=NN),
        out_shape=jax.ShapeDtypeStruct((2, EPAD), jnp.int32),
    )(edge_index)


def _h2_of(a, w1_ref, b1_ref, w2_ref, H):
    acc = jnp.zeros_like(a)
    for k in range(H):
        w1k = w1_ref[0:1, k:k + 1]
        b1k = b1_ref[0:1, k:k + 1]
        w2k = w2_ref[0:1, k:k + 1]
        acc = acc + w2k * jnp.maximum(a * w1k + b1k, 0.0)
    return acc


def _tc1_body(degp_ref, c0_ref, c1_ref, xs0_ref, xs1_ref, xall_ref,
              w1_ref, b1_ref, w2_ref,
              utab_ref, h2x_ref, dinv_ref, *, NN, H):
    deg = jnp.sum(degp_ref[...], axis=0, keepdims=True) + 1.0
    dinv = lax.rsqrt(deg)[:, :NN]
    dinv_ref[...] = dinv
    dcol = dinv.reshape(NN, 1)
    a0t = dcol * (jax.lax.dot_general(
        c0_ref[...], xs0_ref[...], (((1,), (0,)), ((), ())),
        precision=_HIGH, preferred_element_type=jnp.float32) +
        jax.lax.dot_general(
        c1_ref[...], xs1_ref[...], (((1,), (0,)), ((), ())),
        precision=_HIGH, preferred_element_type=jnp.float32))
    utab = dcol * _h2_of(a0t, w1_ref, b1_ref, w2_ref, H)
    utab_ref[...] = jnp.concatenate(
        [utab, jnp.zeros((8, LANES), jnp.float32)], axis=0)
    h2x_ref[...] = _h2_of(xall_ref[...], w1_ref, b1_ref, w2_ref, H)


def _tc1(degp, c0, c1, xs0, xs1, xall, w1, b1, w2):
    NN = c0.shape[0]
    H = w1.shape[1]
    return pl.pallas_call(
        functools.partial(_tc1_body, NN=NN, H=H),
        out_shape=(jax.ShapeDtypeStruct((NN + 8, LANES), jnp.float32),
                   jax.ShapeDtypeStruct(xall.shape, jnp.float32),
                   jax.ShapeDtypeStruct((1, NN), jnp.float32)),
    )(degp, c0, c1, xs0, xs1, xall, w1, b1, w2)


def _gru_unrolled(gi, Whh_t, bhh, B, T, HG):
    h = jnp.zeros((B, HG), jnp.float32)
    outs = []
    for t in range(T):
        gh = jnp.dot(h, Whh_t, preferred_element_type=jnp.float32) + bhh
        g = gi[:, t, :]
        r = jax.nn.sigmoid(g[:, :HG] + gh[:, :HG])
        z = jax.nn.sigmoid(g[:, HG:2 * HG] + gh[:, HG:2 * HG])
        n = jnp.tanh(g[:, 2 * HG:] + r * gh[:, 2 * HG:])
        h = (1.0 - z) * n + z * h
        outs.append(h)
    return outs


def _tc2_body(agg0_ref, agg1_ref, utab_ref, dinv_ref, h2x_ref, wih0_ref, wsens_ref,
              b1_ref, w2_ref, b2_ref, bih0_ref, whh0_ref, bhh0_ref,
              wih1_ref, whh1_ref, bih1_ref, bhh1_ref, fcw_ref, fcb_ref,
              out_ref, *, B, T, NN, S, HG, H):
    b2 = b2_ref[0:1, 0:1]
    utab = utab_ref[...][:NN, :]
    aggu = agg0_ref[...] + agg1_ref[...]
    dcol = dinv_ref[...].reshape(NN, 1)
    out2t = dcol * (aggu + utab) + b2

    wih0 = wih0_ref[...]
    git = jax.lax.dot_general(
        wih0, out2t, (((1,), (0,)), ((), ())),
        precision=_HIGH, preferred_element_type=jnp.float32)
    gi_b0 = git.T[:T, :]

    c0 = jnp.zeros((1, 1), jnp.float32)
    for k in range(H):
        c0 = c0 + w2_ref[0:1, k:k + 1] * jnp.maximum(b1_ref[0:1, k:k + 1], 0.0)
    cc = c0 + b2

    rs = jnp.sum(wih0, axis=1)[None, :]
    h2b = h2x_ref[...][T:, :]
    gi_rest = jax.lax.dot_general(
        h2b + (b2 - cc), wsens_ref[...], (((1,), (1,)), ((), ())),
        precision=_HIGH, preferred_element_type=jnp.float32) + cc * rs

    bih0 = bih0_ref[...]
    gi0 = jnp.concatenate([gi_b0, gi_rest], axis=0) + bih0
    gi0 = gi0.reshape(B, T, 3 * HG)

    o0 = _gru_unrolled(gi0, whh0_ref[...].T, bhh0_ref[...], B, T, HG)
    gi1 = jnp.concatenate([o[:, None, :] for o in o0], axis=1)
    gi1 = jax.lax.dot_general(
        gi1.reshape(B * T, HG), wih1_ref[...], (((1,), (1,)), ((), ())),
        precision=_HIGH,
        preferred_element_type=jnp.float32).reshape(B, T, 3 * HG) + bih1_ref[...]
    o1 = _gru_unrolled(gi1, whh1_ref[...].T, bhh1_ref[...], B, T, HG)
    last = o1[-1]

    out_ref[...] = jax.lax.dot_general(
        last, fcw_ref[...], (((1,), (1,)), ((), ())),
        precision=_HIGH, preferred_element_type=jnp.float32) + fcb_ref[...]


def _tc2(agg0, agg1, utab, dinv, h2x, Wih0, Wsens, b1, w2, b2, bih0, Whh0, bhh0,
         Wih1, Whh1, bih1, bhh1, fcW, fcb):
    NN = Wih0.shape[1]
    T = 12
    B = h2x.shape[0] // T
    S = h2x.shape[1]
    HG = Whh0.shape[1]
    H = w2.shape[1]
    return pl.pallas_call(
        functools.partial(_tc2_body, B=B, T=T, NN=NN, S=S, HG=HG, H=H),
        out_shape=jax.ShapeDtypeStruct((B, NN), jnp.float32),
    )(agg0, agg1, utab, dinv, h2x, Wih0, Wsens, b1, w2, b2, bih0, Whh0, bhh0,
      Wih1, Whh1, bih1, bhh1, fcW, fcb)


def kernel(x_sequence, edge_index, sensor_idx, W1, b1, W2, b2,
           Wih0, Whh0, bih0, bhh0, Wih1, Whh1, bih1, bhh1, fcW, fcb):
    B, T, S = x_sequence.shape
    NN = fcb.shape[0]
    E = edge_index.shape[1]

    EPAD = ((E + NC * NS * 2048 - 1) // (NC * NS * 2048)) * (NC * NS * 2048)
    eip = _pad_edges(edge_index, EPAD, NN)
    srcp = eip[0]
    dstp = eip[1]
    degp, cpart = _sc1(srcp, dstp, sensor_idx, NN, S)
    degp = degp.reshape(NS, NN + LANES)

    degS = jnp.sum(degp[:, sensor_idx], axis=0) + 1.0
    dinvS = lax.rsqrt(degS)
    x0p = jnp.pad(x_sequence[0], ((0, LANES - T), (0, 0)))
    xs = x0p.T * dinvS[:, None]
    xall = x_sequence.reshape(B * T, S)

    cpart = cpart.reshape(NC, NN, S // 2)
    utab, h2x, dinv = _tc1(degp, cpart[0], cpart[1],
                           xs[:S // 2], xs[S // 2:], xall,
                           W1, b1.reshape(1, -1), W2.reshape(1, -1))

    aggp = _sc2(srcp, dstp, utab, NN)
    aggp = aggp.reshape(NC, 10240, LANES)[:, :NN, :]

    Wsens = jnp.take(Wih0, sensor_idx, axis=1)
    return _tc2(aggp[0], aggp[1], utab, dinv.reshape(NN, 1), h2x, Wih0, Wsens,
                b1.reshape(1, -1), W2.reshape(1, -1), b2.reshape(1, 1),
                bih0.reshape(1, -1), Whh0, bhh0.reshape(1, -1),
                Wih1, Whh1, bih1.reshape(1, -1), bhh1.reshape(1, -1),
                fcW, fcb.reshape(1, -1))


# speedup vs baseline: 206.4849x; 1.0418x over previous
import functools

import jax
import jax.numpy as jnp
from jax import lax
from jax.experimental import pallas as pl
from jax.experimental.pallas import tpu as pltpu
from jax.experimental.pallas import tpu_sc as plsc

NC = 2
NS = 16
LANES = 16

_HIGH = jax.lax.Precision.HIGHEST


def _mesh():
    return plsc.VectorSubcoreMesh(
        core_axis_name="c", subcore_axis_name="s", num_cores=NC,
        num_subcores=NS)


def _sc1_body(src_hbm, dst_hbm, sens_hbm, degp_hbm, cpart_hbm,
              srcb, dstb, invb, degb, flat1, flat2, ones2, zb, sensb, c_sh,
              *, NN, S, EPT, CH, NWIN, WIN, DEGN):
    c = lax.axis_index("c")
    s = lax.axis_index("s")
    base = s * EPT
    SH = S // 2
    CSZ = NN * SH
    CT = CSZ // NS

    neg1 = jnp.full((LANES,), -1, jnp.int32)
    zi = jnp.zeros((LANES,), jnp.int32)
    zf = jnp.zeros((LANES,), jnp.float32)
    onef = jnp.ones((LANES,), jnp.float32)

    def ini(i, _):
        invb[pl.ds(i * LANES, LANES)] = neg1
        return 0
    lax.fori_loop(0, (NN + LANES) // LANES, ini, 0)

    def ini2(i, _):
        flat1[pl.ds(i * LANES, LANES)] = zi
        return 0
    lax.fori_loop(0, (NWIN * WIN) // LANES, ini2, 0)

    def ini3(i, _):
        degb[pl.ds(i * LANES, LANES)] = zf
        return 0
    lax.fori_loop(0, DEGN // LANES, ini3, 0)

    def ini4(i, _):
        zb[pl.ds(i * LANES, LANES)] = zf
        return 0
    lax.fori_loop(0, 4000 // LANES, ini4, 0)

    for g in range(WIN // LANES):
        ones2[0, pl.ds(g * LANES, LANES)] = onef

    for j in range(CT // 4000):
        pltpu.sync_copy(zb, c_sh.at[pl.ds(s * CT + j * 4000, 4000)])

    pltpu.sync_copy(sens_hbm, sensb)
    for g in range(S // LANES):
        sidx = sensb[pl.ds(g * LANES, LANES)]
        plsc.store_scatter(invb, [sidx],
                           lax.iota(jnp.int32, LANES) + g * LANES)

    plsc.subcore_barrier()

    koff = c * SH

    def chunk(i, off):
        pltpu.sync_copy(src_hbm.at[pl.ds(base + i * CH, CH)], srcb)
        pltpu.sync_copy(dst_hbm.at[pl.ds(base + i * CH, CH)], dstb)

        def grp(g, off):
            s16 = srcb[pl.ds(g * LANES, LANES)]
            d16 = dstb[pl.ds(g * LANES, LANES)]
            plsc.addupdate_scatter(degb, [d16], onef)
            k16 = plsc.load_gather(invb, [s16]) - koff
            m = (k16 >= 0) & (k16 < SH)
            flat = d16 * SH + k16
            plsc.store_compressed(flat1.at[pl.ds(off, LANES)], flat, mask=m)
            return off + jnp.sum(jnp.where(m, 1, 0))
        return lax.fori_loop(0, CH // LANES, grp, off)
    cnt = lax.fori_loop(0, EPT // CH, chunk, jnp.int32(0))

    nwin = (cnt + (WIN - 1)) // WIN

    def towin(w, _):
        for g in range(WIN // LANES):
            flat2[w, pl.ds(g * LANES, LANES)] = \
                flat1[pl.ds(w * WIN + g * LANES, LANES)]
        return 0
    lax.fori_loop(0, nwin, towin, 0)

    tail = cnt - (nwin - 1) * WIN
    for g in range(WIN // LANES):
        ones2[1, pl.ds(g * LANES, LANES)] = jnp.where(
            lax.iota(jnp.int32, LANES) + g * LANES < tail, 1.0, 0.0)

    def scat(w, _):
        row = jnp.where(w < nwin - 1, 0, 1)
        pltpu.sync_copy(ones2.at[row], c_sh.at[flat2.at[w]], add=True)
        return 0
    lax.fori_loop(0, nwin, scat, 0)

    @pl.when(s == 0)
    def _():
        for g in range(WIN // LANES):
            if g < SH // LANES:
                sidx = sensb[pl.ds(koff + g * LANES, LANES)]
                flat2[0, pl.ds(g * LANES, LANES)] = \
                    sidx * SH + lax.iota(jnp.int32, LANES) + g * LANES
                ones2[1, pl.ds(g * LANES, LANES)] = onef
            else:
                flat2[0, pl.ds(g * LANES, LANES)] = zi
                ones2[1, pl.ds(g * LANES, LANES)] = zf
        pltpu.sync_copy(ones2.at[1], c_sh.at[flat2.at[0]], add=True)

    plsc.subcore_barrier()

    @pl.when(c == 0)
    def _():
        pltpu.sync_copy(degb, degp_hbm.at[pl.ds(s * DEGN, DEGN)])
    for j in range(CT // 4000):
        pltpu.sync_copy(c_sh.at[pl.ds(s * CT + j * 4000, 4000)], zb)
        pltpu.sync_copy(zb, cpart_hbm.at[pl.ds(c * CSZ + s * CT + j * 4000,
                                               4000)])


def _sc1(srcp, dstp, sensor_idx, NN, S):
    E = srcp.shape[0]
    EPT = E // NS
    CH = 4096
    WIN = 128
    NWIN = EPT // WIN
    DEGN = NN + LANES
    body = functools.partial(_sc1_body, NN=NN, S=S, EPT=EPT, CH=CH,
                             NWIN=NWIN, WIN=WIN, DEGN=DEGN)
    f = pl.kernel(
        body,
        out_type=(jax.ShapeDtypeStruct((NS * DEGN,), jnp.float32),
                  jax.ShapeDtypeStruct((NN * S,), jnp.float32)),
        mesh=_mesh(),
        compiler_params=pltpu.CompilerParams(needs_layout_passes=False),
        scratch_types=[
            pltpu.VMEM((CH,), jnp.int32),
            pltpu.VMEM((CH,), jnp.int32),
            pltpu.VMEM((NN + LANES,), jnp.int32),
            pltpu.VMEM((DEGN,), jnp.float32),
            pltpu.VMEM((NWIN * WIN,), jnp.int32),
            pltpu.VMEM((NWIN, WIN), jnp.int32),
            pltpu.VMEM((2, WIN), jnp.float32),
            pltpu.VMEM((4000,), jnp.float32),
            pltpu.VMEM((S,), jnp.int32),
            pltpu.VMEM_SHARED((NN * S // 2,), jnp.float32),
        ],
    )
    return f(srcp, dstp, sensor_idx)


def _sc2_body(src_hbm, dst_hbm, utab_hbm, aggp_hbm,
              srcb, dstb, rows, zrows, agg_sh, semL, semG, semS,
              *, NN, NNP, EPT, CH, WIN):
    c = lax.axis_index("c")
    s = lax.axis_index("s")
    wid = c * NS + s
    RPT = NNP // NS

    zf = jnp.zeros((LANES,), jnp.float32)

    def ini(i, _):
        zrows[i] = zf
        return 0
    lax.fori_loop(0, RPT, ini, 0)
    pltpu.sync_copy(zrows, agg_sh.at[pl.ds(s * RPT, RPT)])
    plsc.subcore_barrier()

    NCH = EPT // CH
    NBUF = 3
    tbase = wid * EPT
    pend_scat = [None] * NBUF
    loads = [None] * NBUF

    def issue_load(i):
        b = i % NBUF
        if pend_scat[b] is not None:
            pend_scat[b].wait()
            pend_scat[b] = None
        loads[b] = (
            pltpu.async_copy(src_hbm.at[pl.ds(tbase + i * CH, CH)],
                             srcb[b], semL[b]),
            pltpu.async_copy(dst_hbm.at[pl.ds(tbase + i * CH, CH)],
                             dstb[b], semL[b]))

    issue_load(0)
    if NCH > 1:
        issue_load(1)
    for i in range(NCH):
        b = i % NBUF
        loads[b][0].wait()
        loads[b][1].wait()
        pltpu.async_copy(utab_hbm.at[srcb[b]], rows[b], semG[b]).wait()
        pend_scat[b] = pltpu.async_copy(rows[b], agg_sh.at[dstb[b]],
                                        semS[b], add=True)
        if i + 2 < NCH:
            issue_load(i + 2)
    for b in range(NBUF):
        if pend_scat[b] is not None:
            pend_scat[b].wait()

    plsc.subcore_barrier()
    pltpu.sync_copy(agg_sh.at[pl.ds(s * RPT, RPT)], zrows)
    pltpu.sync_copy(zrows, aggp_hbm.at[c, s])


def _sc2(srcp, dst2, utab, NN):
    E = srcp.shape[0]
    EPT = E // (NC * NS)
    CH = 1024
    WIN = 128
    NNP = 10240
    body = functools.partial(_sc2_body, NN=NN, NNP=NNP, EPT=EPT, CH=CH,
                             WIN=WIN)
    f = pl.kernel(
        body,
        out_type=jax.ShapeDtypeStruct((NC, NS, NNP // NS, LANES),
                                      jnp.float32),
        mesh=_mesh(),
        compiler_params=pltpu.CompilerParams(needs_layout_passes=False,
                                             use_tc_tiling_on_sc=False),
        scratch_types=[
            [pltpu.VMEM((CH,), jnp.int32) for _ in range(3)],
            [pltpu.VMEM((CH,), jnp.int32) for _ in range(3)],
            [pltpu.VMEM((CH, LANES), jnp.float32) for _ in range(3)],
            pltpu.VMEM((NNP // NS, LANES), jnp.float32),
            pltpu.VMEM_SHARED((NNP, LANES), jnp.float32),
            [pltpu.SemaphoreType.DMA for _ in range(3)],
            [pltpu.SemaphoreType.DMA for _ in range(3)],
            [pltpu.SemaphoreType.DMA for _ in range(3)],
        ],
    )
    return f(srcp, dst2, utab)


def _pad_body(ei_ref, src_ref, dst_ref, *, E, EPAD, NN):
    ei = ei_ref[...]
    pad = jnp.full((1, EPAD - E), NN, jnp.int32)
    src_ref[...] = jnp.concatenate([ei[0:1, :], pad], axis=1)
    dst_ref[...] = jnp.concatenate([ei[1:2, :], pad], axis=1)


def _pad_edges(edge_index, EPAD, NN):
    E = edge_index.shape[1]
    return pl.pallas_call(
        functools.partial(_pad_body, E=E, EPAD=EPAD, NN=NN),
        out_shape=(jax.ShapeDtypeStruct((1, EPAD), jnp.int32),
                   jax.ShapeDtypeStruct((1, EPAD), jnp.int32)),
    )(edge_index)


def _h2_of(a, w1_ref, b1_ref, w2_ref, H):
    acc = jnp.zeros_like(a)
    for k in range(H):
        w1k = w1_ref[0:1, k:k + 1]
        b1k = b1_ref[0:1, k:k + 1]
        w2k = w2_ref[0:1, k:k + 1].astype(jnp.bfloat16).astype(jnp.float32)
        r = jnp.maximum(a * w1k + b1k, 0.0)
        rb = r.astype(jnp.bfloat16).astype(jnp.float32)
        acc = acc + rb * w2k
    return acc


def _tc1_body(degp_ref, c0_ref, c1_ref, xs0_ref, xs1_ref, xall_ref,
              w1_ref, b1_ref, w2_ref,
              utab_ref, h2x_ref, dinv_ref, *, NN, H):
    deg = jnp.sum(degp_ref[...], axis=0, keepdims=True) + 1.0
    dinv = lax.rsqrt(deg)[:, :NN]
    dinv_ref[...] = dinv
    dcol = dinv.reshape(NN, 1)
    a0t = dcol * (jax.lax.dot_general(
        c0_ref[...], xs0_ref[...], (((1,), (0,)), ((), ())),
        precision=_HIGH, preferred_element_type=jnp.float32) +
        jax.lax.dot_general(
        c1_ref[...], xs1_ref[...], (((1,), (0,)), ((), ())),
        precision=_HIGH, preferred_element_type=jnp.float32))
    utab = dcol * _h2_of(a0t, w1_ref, b1_ref, w2_ref, H)
    utab_ref[...] = jnp.concatenate(
        [utab, jnp.zeros((8, LANES), jnp.float32)], axis=0)
    h2x_ref[...] = _h2_of(xall_ref[...], w1_ref, b1_ref, w2_ref, H)


def _tc1(degp, c0, c1, xs0, xs1, xall, w1, b1, w2):
    NN = c0.shape[0]
    H = w1.shape[1]
    return pl.pallas_call(
        functools.partial(_tc1_body, NN=NN, H=H),
        out_shape=(jax.ShapeDtypeStruct((NN + 8, LANES), jnp.float32),
                   jax.ShapeDtypeStruct(xall.shape, jnp.float32),
                   jax.ShapeDtypeStruct((1, NN), jnp.float32)),
    )(degp, c0, c1, xs0, xs1, xall, w1, b1, w2)


def _gru_unrolled(gi, Whh_t, bhh, B, T, HG):
    h = jnp.zeros((B, HG), jnp.float32)
    outs = []
    for t in range(T):
        gh = jnp.dot(h, Whh_t, preferred_element_type=jnp.float32) + bhh
        g = gi[:, t, :]
        r = jax.nn.sigmoid(g[:, :HG] + gh[:, :HG])
        z = jax.nn.sigmoid(g[:, HG:2 * HG] + gh[:, HG:2 * HG])
        n = jnp.tanh(g[:, 2 * HG:] + r * gh[:, 2 * HG:])
        h = (1.0 - z) * n + z * h
        outs.append(h)
    return outs


def _tc2_body(agg0_ref, agg1_ref, utab_ref, dinv_ref, h2x_ref, wih0_ref, wsens_ref,
              b1_ref, w2_ref, b2_ref, bih0_ref, whh0_ref, bhh0_ref,
              wih1_ref, whh1_ref, bih1_ref, bhh1_ref, fcw_ref, fcb_ref,
              out_ref, *, B, T, NN, S, HG, H):
    b2 = b2_ref[0:1, 0:1]
    utab = utab_ref[...][:NN, :]
    aggu = (agg0_ref[...] + agg1_ref[...])[:NN, :]
    dcol = dinv_ref[...].reshape(NN, 1)
    out2t = dcol * (aggu + utab) + b2

    wih0 = wih0_ref[...]
    git = jax.lax.dot_general(
        wih0, out2t, (((1,), (0,)), ((), ())),
        preferred_element_type=jnp.float32)
    gi_b0 = git.T[:T, :]

    c0 = jnp.zeros((1, 1), jnp.float32)
    for k in range(H):
        c0 = c0 + w2_ref[0:1, k:k + 1] * jnp.maximum(b1_ref[0:1, k:k + 1], 0.0)
    cc = c0 + b2

    rs = jnp.sum(wih0, axis=1)[None, :]
    h2b = h2x_ref[...][T:, :]
    gi_rest = jax.lax.dot_general(
        h2b + (b2 - cc), wsens_ref[...], (((1,), (1,)), ((), ())),
        preferred_element_type=jnp.float32) + cc * rs

    bih0 = bih0_ref[...]
    gi0 = jnp.concatenate([gi_b0, gi_rest], axis=0) + bih0
    gi0 = gi0.reshape(B, T, 3 * HG)

    o0 = _gru_unrolled(gi0, whh0_ref[...].T, bhh0_ref[...], B, T, HG)
    gi1 = jnp.concatenate([o[:, None, :] for o in o0], axis=1)
    gi1 = jax.lax.dot_general(
        gi1.reshape(B * T, HG), wih1_ref[...], (((1,), (1,)), ((), ())),
        preferred_element_type=jnp.float32).reshape(B, T, 3 * HG) + bih1_ref[...]
    o1 = _gru_unrolled(gi1, whh1_ref[...].T, bhh1_ref[...], B, T, HG)
    last = o1[-1]

    out_ref[...] = jax.lax.dot_general(
        last, fcw_ref[...], (((1,), (1,)), ((), ())),
        preferred_element_type=jnp.float32) + fcb_ref[...]


def _tc2(agg0, agg1, utab, dinv, h2x, Wih0, Wsens, b1, w2, b2, bih0, Whh0, bhh0,
         Wih1, Whh1, bih1, bhh1, fcW, fcb):
    NN = Wih0.shape[1]
    T = 12
    B = h2x.shape[0] // T
    S = h2x.shape[1]
    HG = Whh0.shape[1]
    H = w2.shape[1]
    return pl.pallas_call(
        functools.partial(_tc2_body, B=B, T=T, NN=NN, S=S, HG=HG, H=H),
        out_shape=jax.ShapeDtypeStruct((B, NN), jnp.float32),
    )(agg0, agg1, utab, dinv, h2x, Wih0, Wsens, b1, w2, b2, bih0, Whh0, bhh0,
      Wih1, Whh1, bih1, bhh1, fcW, fcb)


def kernel(x_sequence, edge_index, sensor_idx, W1, b1, W2, b2,
           Wih0, Whh0, bih0, bhh0, Wih1, Whh1, bih1, bhh1, fcW, fcb):
    B, T, S = x_sequence.shape
    NN = fcb.shape[0]
    E = edge_index.shape[1]

    EPAD = ((E + NC * NS * 2048 - 1) // (NC * NS * 2048)) * (NC * NS * 2048)
    srcp2, dstp2 = _pad_edges(edge_index, EPAD, NN)
    srcp = srcp2.reshape(EPAD)
    dstp = dstp2.reshape(EPAD)
    degp, cpart = _sc1(srcp, dstp, sensor_idx, NN, S)
    degp = degp.reshape(NS, NN + LANES)

    degS = jnp.sum(degp[:, sensor_idx], axis=0) + 1.0
    dinvS = lax.rsqrt(degS)
    x0p = jnp.pad(x_sequence[0], ((0, LANES - T), (0, 0)))
    xs = x0p.T * dinvS[:, None]
    xall = x_sequence.reshape(B * T, S)

    cpart = cpart.reshape(NC, NN, S // 2)
    utab, h2x, dinv = _tc1(degp, cpart[0], cpart[1],
                           xs[:S // 2], xs[S // 2:], xall,
                           W1, b1.reshape(1, -1), W2.reshape(1, -1))

    aggp = _sc2(srcp, dstp, utab, NN)
    aggp = aggp.reshape(NC, 10240, LANES)

    Wsens = jnp.take(Wih0, sensor_idx, axis=1)
    return _tc2(aggp[0], aggp[1], utab, dinv.reshape(NN, 1), h2x, Wih0, Wsens,
                b1.reshape(1, -1), W2.reshape(1, -1), b2.reshape(1, 1),
                bih0.reshape(1, -1), Whh0, bhh0.reshape(1, -1),
                Wih1, Whh1, bih1.reshape(1, -1), bhh1.reshape(1, -1),
                fcW, fcb.reshape(1, -1))
